# baseline (speedup 1.0000x reference)
import functools

import jax
import jax.numpy as jnp
from jax import lax
from jax.experimental import pallas as pl
from jax.experimental.pallas import tpu as pltpu
from jax.experimental.pallas import tpu_sc as plsc

_N_TABLE = 1000000
_D = 32
_B = 16384


@functools.lru_cache(maxsize=None)
def _build_gather():
    info = plsc.get_sparse_core_info()
    nc = info.num_cores

    mesh = plsc.VectorSubcoreMesh(core_axis_name="c", subcore_axis_name="s")

    @functools.partial(
        pl.kernel,
        mesh=mesh,
        out_type=jax.ShapeDtypeStruct((_D, _B), jnp.float32),
        scratch_types=[
            pltpu.VMEM((_B,), jnp.int32),
            pltpu.VMEM((_B,), jnp.float32),
            pltpu.SemaphoreType.DMA,
        ],
        compiler_params=pltpu.CompilerParams(use_tc_tiling_on_sc=False),
    )
    def gather(idx_hbm, tab_t_hbm, out_t_hbm, idx_v, col_v, sem):
        wid = lax.axis_index("s") * nc + lax.axis_index("c")
        pltpu.sync_copy(idx_hbm, idx_v)
        pltpu.async_copy(tab_t_hbm.at[wid].at[idx_v], col_v, sem).wait()
        pltpu.sync_copy(col_v, out_t_hbm.at[wid])

    return gather


def kernel(instance_ids, embedding_instance_weight):
    gather = _build_gather()
    out_t = gather(instance_ids.astype(jnp.int32), embedding_instance_weight.T)
    return out_t.T


# speedup vs baseline: 7.1893x; 7.1893x over previous
import functools

import jax
import jax.numpy as jnp
from jax import lax
from jax.experimental import pallas as pl
from jax.experimental.pallas import tpu as pltpu
from jax.experimental.pallas import tpu_sc as plsc

_V = 1000000
_D = 32
_B = 16384
_NW = (_V + 127) // 128
_WPT = (_NW + 31) // 32
_WPAIR = (_WPT + 1) // 2


def _iota16():
    return lax.iota(jnp.int32, 16)


def _fill(x):
    return jnp.full((16,), x, jnp.int32)


@functools.lru_cache(maxsize=None)
def _build_k1():
    info = plsc.get_sparse_core_info()
    nc = info.num_cores
    mesh = plsc.VectorSubcoreMesh(core_axis_name="c", subcore_axis_name="s")

    @functools.partial(
        pl.kernel,
        mesh=mesh,
        out_type=(
            jax.ShapeDtypeStruct((32 * _B * _D,), jnp.float32),
            jax.ShapeDtypeStruct((32 * _B,), jnp.int32),
            jax.ShapeDtypeStruct((32 * 128,), jnp.int32),
        ),
        scratch_types=[
            pltpu.VMEM((_B,), jnp.int32),
            pltpu.VMEM((_B,), jnp.int32),
            pltpu.VMEM((_B,), jnp.int32),
            pltpu.VMEM((_B,), jnp.int32),
            pltpu.VMEM((_D, 128), jnp.float32),
            pltpu.VMEM((_D, 128), jnp.float32),
            pltpu.VMEM((128,), jnp.float32),
            pltpu.VMEM((16,), jnp.int32),
            pltpu.VMEM((16,), jnp.int32),
            pltpu.VMEM((128,), jnp.int32),
            pltpu.SemaphoreType.DMA,
            pltpu.SemaphoreType.DMA,
        ],
        compiler_params=pltpu.CompilerParams(needs_layout_passes=False),
    )
    def k1(idx_hbm, tab_t_hbm, rows_hbm, pos_hbm, cnt_hbm,
           idx_v, pid, ppos, pext, win_a, win_b, rstage, tmp_id, tmp_pos, cvec,
           sem_a, sem_b):
        wid = lax.axis_index("s") * nc + lax.axis_index("c")
        wb = wid * _WPT
        region = wid * (_B * _D)

        pltpu.sync_copy(idx_hbm, idx_v)

        def fbody(c, off):
            lanes = c * 16 + _iota16()
            v = plsc.load_gather(idx_v, [lanes])
            w = v >> 7
            m = (w >= wb) & (w < wb + _WPT)
            s = plsc.cumsum(m.astype(jnp.int32))
            dest = off + s - 1
            plsc.store_scatter(pid, [dest], v, mask=m)
            plsc.store_scatter(ppos, [dest], lanes, mask=m)
            return off + s[15]

        cnt = lax.fori_loop(0, _B // 16, fbody, jnp.int32(0))
        nch = (cnt + 15) // 16

        def win_off(wg):
            return pl.multiple_of(jnp.minimum(wg, _NW - 1) * 128, 128)

        def fetch(wg, buf, sem):
            pltpu.async_copy(tab_t_hbm.at[:, pl.ds(win_off(wg), 128)], buf, sem)

        def wait(buf, sem):
            pltpu.make_async_copy(
                tab_t_hbm.at[:, pl.ds(0, 128)], buf, sem
            ).wait()

        def process(k, buf, s0):
            wg = wb + k
            live = (k < _WPT) & (wg < _NW)
            off_lane = jnp.minimum(wg, _NW - 1) * 128
            n_loc = jnp.where(live, nch, 0)

            def rb(c2, s1):
                lanes = c2 * 16 + _iota16()
                vid = plsc.load_gather(pid, [lanes])
                vpos = plsc.load_gather(ppos, [lanes])
                m = ((vid >> 7) == wg) & (lanes < cnt)
                sc = plsc.cumsum(m.astype(jnp.int32))
                tot = sc[15]
                plsc.store_scatter(tmp_id, [sc - 1], vid, mask=m)
                plsc.store_scatter(tmp_pos, [sc - 1], vpos, mask=m)

                def eb(k2, s2):
                    lane = plsc.load_gather(tmp_id, [_fill(k2)])[0] - off_lane
                    posv = plsc.load_gather(tmp_pos, [_fill(k2)])[0]
                    r16 = _iota16()
                    g0 = plsc.load_gather(buf, [r16, _fill(lane)])
                    g1 = plsc.load_gather(buf, [r16 + 16, _fill(lane)])
                    slot = (s2 & 3) * 32
                    plsc.store_scatter(rstage, [slot + r16], g0)
                    plsc.store_scatter(rstage, [slot + 16 + r16], g1)
                    plsc.store_scatter(
                        pext, [_fill(s2)], _fill(posv), mask=_iota16() < 1
                    )

                    @pl.when((s2 & 3) == 3)
                    def _():
                        dst = pl.multiple_of(region + (s2 - 3) * _D, 128)
                        pltpu.sync_copy(rstage, rows_hbm.at[pl.ds(dst, 128)])

                    return s2 + 1

                return lax.fori_loop(0, tot, eb, s1)

            return lax.fori_loop(0, n_loc, rb, s0)

        fetch(wb, win_a, sem_a)

        def outer(i, s):
            k0 = i * 2
            wait(win_a, sem_a)
            fetch(wb + k0 + 1, win_b, sem_b)
            s = process(k0, win_a, s)
            wait(win_b, sem_b)
            fetch(wb + k0 + 2, win_a, sem_a)
            s = process(k0 + 1, win_b, s)
            return s

        s = lax.fori_loop(0, _WPAIR, outer, jnp.int32(0))
        wait(win_a, sem_a)

        @pl.when((s & 3) != 0)
        def _():
            dst = pl.multiple_of(region + (s >> 2) * 128, 128)
            pltpu.sync_copy(rstage, rows_hbm.at[pl.ds(dst, 128)])

        pltpu.sync_copy(pext, pos_hbm.at[pl.ds(wid * _B, _B)])

        for j in range(8):
            plsc.store_scatter(cvec, [j * 16 + _iota16()], _fill(s))
        pltpu.sync_copy(cvec, cnt_hbm.at[pl.ds(wid * 128, 128)])

    return k1


@functools.lru_cache(maxsize=None)
def _build_k2():
    info = plsc.get_sparse_core_info()
    nc = info.num_cores
    mesh = plsc.VectorSubcoreMesh(core_axis_name="c", subcore_axis_name="s")

    @functools.partial(
        pl.kernel,
        mesh=mesh,
        out_type=jax.ShapeDtypeStruct((_B, _D), jnp.float32),
        scratch_types=[
            pltpu.VMEM((128,), jnp.int32),
            pltpu.VMEM((64,), jnp.int32),
            pltpu.VMEM((64, _D), jnp.float32),
            pltpu.SemaphoreType.DMA,
        ],
        compiler_params=pltpu.CompilerParams(
            use_tc_tiling_on_sc=False, needs_layout_passes=False
        ),
    )
    def k2(cnt_hbm, pos_hbm, rows_hbm, out_hbm, cvec, idx64, stage, sem):
        wid = lax.axis_index("s") * nc + lax.axis_index("c")
        pltpu.sync_copy(cnt_hbm.at[pl.ds(wid * 128, 128)], cvec)
        cnt = plsc.load_gather(cvec, [_fill(0)])[0]
        nch = (cnt + 63) // 64

        def cb(c, _):
            row0 = wid * _B + c * 64
            pltpu.sync_copy(pos_hbm.at[pl.ds(row0, 64)], idx64)
            pltpu.sync_copy(rows_hbm.at[pl.ds(row0, 64), :], stage)
            rem = cnt - c * 64

            @pl.when(rem < 64)
            def _():
                p0 = plsc.load_gather(idx64, [_fill(0)])[0]
                for j in range(4):
                    lanes = j * 16 + _iota16()
                    vv = plsc.load_gather(idx64, [lanes])
                    sel = jnp.where(lanes >= rem, _fill(p0), vv)
                    plsc.store_scatter(idx64, [lanes], sel)
                r0a = plsc.load_gather(stage, [_fill(0), _iota16()])
                r0b = plsc.load_gather(stage, [_fill(0), _iota16() + 16])
                for r in range(1, 64):
                    @pl.when(r >= rem)
                    def _():
                        plsc.store_scatter(stage, [_fill(r), _iota16()], r0a)
                        plsc.store_scatter(stage, [_fill(r), _iota16() + 16], r0b)

            pltpu.async_copy(stage, out_hbm.at[idx64], sem).wait()
            return ()

        lax.fori_loop(0, nch, cb, ())

    return k2


def kernel(instance_ids, embedding_instance_weight):
    k1 = _build_k1()
    k2 = _build_k2()
    rows1, pos1, cnts = k1(
        instance_ids.astype(jnp.int32), embedding_instance_weight.T
    )
    rows2 = rows1.reshape(32 * _B, _D)
    return k2(cnts, pos1, rows2)


# speedup vs baseline: 17.9772x; 2.5006x over previous
import functools

import jax
import jax.numpy as jnp
from jax import lax
from jax.experimental import pallas as pl
from jax.experimental.pallas import tpu as pltpu
from jax.experimental.pallas import tpu_sc as plsc

_V = 1000000
_D = 32
_B = 16384
_NW = (_V + 127) // 128
_WPT = (_NW + 31) // 32
_WPAIR = (_WPT + 1) // 2


def _iota16():
    return lax.iota(jnp.int32, 16)


def _fill(x):
    return jnp.full((16,), x, jnp.int32)


@functools.lru_cache(maxsize=None)
def _build_k1():
    info = plsc.get_sparse_core_info()
    nc = info.num_cores
    mesh = plsc.VectorSubcoreMesh(core_axis_name="c", subcore_axis_name="s")

    @functools.partial(
        pl.kernel,
        mesh=mesh,
        out_type=(
            jax.ShapeDtypeStruct((32 * _B * _D,), jnp.float32),
            jax.ShapeDtypeStruct((32 * _B,), jnp.int32),
            jax.ShapeDtypeStruct((32 * 128,), jnp.int32),
        ),
        scratch_types=[
            pltpu.VMEM((_B,), jnp.int32),
            pltpu.VMEM((_B,), jnp.int32),
            pltpu.VMEM((_B,), jnp.int32),
            pltpu.VMEM((_D, 1024), jnp.float32),
            pltpu.VMEM((_D, 1024), jnp.float32),
            pltpu.VMEM((128,), jnp.float32),
            pltpu.VMEM((16,), jnp.int32),
            pltpu.VMEM((16,), jnp.int32),
            pltpu.VMEM((128,), jnp.int32),
            pltpu.SemaphoreType.DMA,
            pltpu.SemaphoreType.DMA,
        ],
        compiler_params=pltpu.CompilerParams(needs_layout_passes=False),
    )
    def k1(idx_hbm, tab_t_hbm, rows_hbm, pos_hbm, cnt_hbm,
           idx_v, pid, ppos, win_a, win_b, rstage, tmp_id, tmp_pos, cvec,
           sem_a, sem_b):
        pext = idx_v
        wid = lax.axis_index("s") * nc + lax.axis_index("c")
        wb = wid * _WPT
        region = wid * (_B * _D)

        pltpu.sync_copy(idx_hbm, idx_v)

        def fbody(c, off):
            lanes = c * 16 + _iota16()
            v = plsc.load_gather(idx_v, [lanes])
            w = v >> 7
            m = (w >= wb) & (w < wb + _WPT)
            s = plsc.cumsum(m.astype(jnp.int32))
            dest = off + s - 1
            plsc.store_scatter(pid, [dest], v, mask=m)
            plsc.store_scatter(ppos, [dest], lanes, mask=m)
            return off + s[15]

        cnt = lax.fori_loop(0, _B // 16, fbody, jnp.int32(0))
        nch = (cnt + 15) // 16

        def win_off(wg):
            return pl.multiple_of(jnp.minimum(wg, _NW - 1) * 128, 128)

        def fetch_group(g, buf, sem):
            wg0 = wb + g * 8
            for j in range(8):
                pltpu.async_copy(
                    tab_t_hbm.at[:, pl.ds(win_off(wg0 + j), 128)],
                    buf.at[:, pl.ds(j * 128, 128)],
                    sem,
                )

        def wait_group(buf, sem):
            pltpu.make_async_copy(
                tab_t_hbm.at[:, pl.ds(0, 1024)], buf, sem
            ).wait()

        def process(g, buf, s0):
            wg0 = wb + g * 8
            base = wg0 * 128

            def rb(c2, s1):
                lanes = c2 * 16 + _iota16()
                vid = plsc.load_gather(pid, [lanes])
                vpos = plsc.load_gather(ppos, [lanes])
                w = vid >> 7
                m = (w >= wg0) & (w < wg0 + 8) & (lanes < cnt)
                sc = plsc.cumsum(m.astype(jnp.int32))
                tot = sc[15]
                plsc.store_scatter(tmp_id, [sc - 1], vid, mask=m)
                plsc.store_scatter(tmp_pos, [sc - 1], vpos, mask=m)

                def eb(k2, s2):
                    lane = plsc.load_gather(tmp_id, [_fill(k2)])[0] - base
                    posv = plsc.load_gather(tmp_pos, [_fill(k2)])[0]
                    r16 = _iota16()
                    g0 = plsc.load_gather(buf, [r16, _fill(lane)])
                    g1 = plsc.load_gather(buf, [r16 + 16, _fill(lane)])
                    slot = (s2 & 3) * 32
                    plsc.store_scatter(rstage, [slot + r16], g0)
                    plsc.store_scatter(rstage, [slot + 16 + r16], g1)
                    plsc.store_scatter(
                        pext, [_fill(s2)], _fill(posv), mask=_iota16() < 1
                    )

                    @pl.when((s2 & 3) == 3)
                    def _():
                        dst = pl.multiple_of(region + (s2 - 3) * _D, 128)
                        pltpu.sync_copy(rstage, rows_hbm.at[pl.ds(dst, 128)])

                    return s2 + 1

                return lax.fori_loop(0, tot, eb, s1)

            return lax.fori_loop(0, nch, rb, s0)

        fetch_group(0, win_a, sem_a)

        def outer(i, s):
            g0 = i * 2
            wait_group(win_a, sem_a)
            fetch_group(g0 + 1, win_b, sem_b)
            s = process(g0, win_a, s)
            wait_group(win_b, sem_b)
            fetch_group(g0 + 2, win_a, sem_a)
            s = process(g0 + 1, win_b, s)
            return s

        s = lax.fori_loop(0, 16, outer, jnp.int32(0))
        wait_group(win_a, sem_a)

        @pl.when((s & 3) != 0)
        def _():
            dst = pl.multiple_of(region + (s >> 2) * 128, 128)
            pltpu.sync_copy(rstage, rows_hbm.at[pl.ds(dst, 128)])

        pltpu.sync_copy(pext, pos_hbm.at[pl.ds(wid * _B, _B)])

        for j in range(8):
            plsc.store_scatter(cvec, [j * 16 + _iota16()], _fill(s))
        pltpu.sync_copy(cvec, cnt_hbm.at[pl.ds(wid * 128, 128)])

    return k1


@functools.lru_cache(maxsize=None)
def _build_k2():
    info = plsc.get_sparse_core_info()
    nc = info.num_cores
    mesh = plsc.VectorSubcoreMesh(core_axis_name="c", subcore_axis_name="s")

    @functools.partial(
        pl.kernel,
        mesh=mesh,
        out_type=jax.ShapeDtypeStruct((_B, _D), jnp.float32),
        scratch_types=[
            pltpu.VMEM((128,), jnp.int32),
            pltpu.VMEM((64,), jnp.int32),
            pltpu.VMEM((64, _D), jnp.float32),
            pltpu.SemaphoreType.DMA,
        ],
        compiler_params=pltpu.CompilerParams(
            use_tc_tiling_on_sc=False, needs_layout_passes=False
        ),
    )
    def k2(cnt_hbm, pos_hbm, rows_hbm, out_hbm, cvec, idx64, stage, sem):
        wid = lax.axis_index("s") * nc + lax.axis_index("c")
        pltpu.sync_copy(cnt_hbm.at[pl.ds(wid * 128, 128)], cvec)
        cnt = plsc.load_gather(cvec, [_fill(0)])[0]
        nch = (cnt + 63) // 64

        def cb(c, _):
            row0 = wid * _B + c * 64
            pltpu.sync_copy(pos_hbm.at[pl.ds(row0, 64)], idx64)
            pltpu.sync_copy(rows_hbm.at[pl.ds(row0, 64), :], stage)
            rem = cnt - c * 64

            @pl.when(rem < 64)
            def _():
                p0 = plsc.load_gather(idx64, [_fill(0)])[0]
                for j in range(4):
                    lanes = j * 16 + _iota16()
                    vv = plsc.load_gather(idx64, [lanes])
                    sel = jnp.where(lanes >= rem, _fill(p0), vv)
                    plsc.store_scatter(idx64, [lanes], sel)
                r0a = plsc.load_gather(stage, [_fill(0), _iota16()])
                r0b = plsc.load_gather(stage, [_fill(0), _iota16() + 16])
                for r in range(1, 64):
                    @pl.when(r >= rem)
                    def _():
                        plsc.store_scatter(stage, [_fill(r), _iota16()], r0a)
                        plsc.store_scatter(stage, [_fill(r), _iota16() + 16], r0b)

            pltpu.async_copy(stage, out_hbm.at[idx64], sem).wait()
            return ()

        lax.fori_loop(0, nch, cb, ())

    return k2


def kernel(instance_ids, embedding_instance_weight):
    k1 = _build_k1()
    k2 = _build_k2()
    rows1, pos1, cnts = k1(
        instance_ids.astype(jnp.int32), embedding_instance_weight.T
    )
    rows2 = rows1.reshape(32 * _B, _D)
    return k2(cnts, pos1, rows2)


# speedup vs baseline: 18.2305x; 1.0141x over previous
import functools

import jax
import jax.numpy as jnp
from jax import lax
from jax.experimental import pallas as pl
from jax.experimental.pallas import tpu as pltpu
from jax.experimental.pallas import tpu_sc as plsc

_V = 1000000
_D = 32
_B = 16384
_NW = (_V + 127) // 128
_WPT = (_NW + 31) // 32
_WPAIR = (_WPT + 1) // 2


def _iota16():
    return lax.iota(jnp.int32, 16)


def _fill(x):
    return jnp.full((16,), x, jnp.int32)


@functools.lru_cache(maxsize=None)
def _build_k1():
    info = plsc.get_sparse_core_info()
    nc = info.num_cores
    mesh = plsc.VectorSubcoreMesh(core_axis_name="c", subcore_axis_name="s")

    @functools.partial(
        pl.kernel,
        mesh=mesh,
        out_type=(
            jax.ShapeDtypeStruct((32 * _B * _D,), jnp.float32),
            jax.ShapeDtypeStruct((32 * _B,), jnp.int32),
            jax.ShapeDtypeStruct((32 * 128,), jnp.int32),
        ),
        scratch_types=[
            pltpu.VMEM((_B,), jnp.int32),
            pltpu.VMEM((_B,), jnp.int32),
            pltpu.VMEM((_B,), jnp.int32),
            pltpu.VMEM((_D, 1024), jnp.float32),
            pltpu.VMEM((_D, 1024), jnp.float32),
            pltpu.VMEM((128,), jnp.float32),
            pltpu.VMEM((16,), jnp.int32),
            pltpu.VMEM((16,), jnp.int32),
            pltpu.VMEM((128,), jnp.int32),
            pltpu.SemaphoreType.DMA,
            pltpu.SemaphoreType.DMA,
        ],
        compiler_params=pltpu.CompilerParams(needs_layout_passes=False),
    )
    def k1(idx_hbm, tab_t_hbm, rows_hbm, pos_hbm, cnt_hbm,
           idx_v, pid, ppos, win_a, win_b, rstage, tmp_id, tmp_pos, cvec,
           sem_a, sem_b):
        pext = idx_v
        wid = lax.axis_index("s") * nc + lax.axis_index("c")
        wb = wid * _WPT
        region = wid * (_B * _D)

        pltpu.sync_copy(idx_hbm, idx_v)

        def fbody(c, off):
            lanes = c * 16 + _iota16()
            v = plsc.load_gather(idx_v, [lanes])
            w = v >> 7
            m = (w >= wb) & (w < wb + _WPT)
            s = plsc.cumsum(m.astype(jnp.int32))
            dest = off + s - 1
            plsc.store_scatter(pid, [dest], v, mask=m)
            plsc.store_scatter(ppos, [dest], lanes, mask=m)
            return off + s[15]

        cnt = lax.fori_loop(0, _B // 16, fbody, jnp.int32(0))
        nch = (cnt + 15) // 16

        def win_off(wg):
            return pl.multiple_of(jnp.minimum(wg, _NW - 1) * 128, 128)

        def fetch_group(g, buf, sem):
            wg0 = wb + g * 8
            for j in range(8):
                pltpu.async_copy(
                    tab_t_hbm.at[:, pl.ds(win_off(wg0 + j), 128)],
                    buf.at[:, pl.ds(j * 128, 128)],
                    sem,
                )

        def wait_group(buf, sem):
            pltpu.make_async_copy(
                tab_t_hbm.at[:, pl.ds(0, 1024)], buf, sem
            ).wait()

        def process(g, buf, s0):
            wg0 = wb + g * 8
            base = wg0 * 128

            def rb(c2, s1):
                lanes = c2 * 16 + _iota16()
                vid = plsc.load_gather(pid, [lanes])
                vpos = plsc.load_gather(ppos, [lanes])
                w = vid >> 7
                m = (w >= wg0) & (w < wg0 + 8) & (lanes < cnt)
                sc = plsc.cumsum(m.astype(jnp.int32))
                tot = sc[15]
                plsc.store_scatter(tmp_id, [sc - 1], vid, mask=m)
                plsc.store_scatter(tmp_pos, [sc - 1], vpos, mask=m)

                def eb(k2, s2):
                    lane = plsc.load_gather(tmp_id, [_fill(k2)])[0] - base
                    posv = plsc.load_gather(tmp_pos, [_fill(k2)])[0]
                    r16 = _iota16()
                    g0 = plsc.load_gather(buf, [r16, _fill(lane)])
                    g1 = plsc.load_gather(buf, [r16 + 16, _fill(lane)])
                    slot = (s2 & 3) * 32
                    plsc.store_scatter(rstage, [slot + r16], g0)
                    plsc.store_scatter(rstage, [slot + 16 + r16], g1)
                    plsc.store_scatter(
                        pext, [_fill(s2)], _fill(posv), mask=_iota16() < 1
                    )

                    @pl.when((s2 & 3) == 3)
                    def _():
                        dst = pl.multiple_of(region + (s2 - 3) * _D, 128)
                        pltpu.sync_copy(rstage, rows_hbm.at[pl.ds(dst, 128)])

                    return s2 + 1

                return lax.fori_loop(0, tot, eb, s1)

            return lax.fori_loop(0, nch, rb, s0)

        fetch_group(0, win_a, sem_a)

        def outer(i, s):
            g0 = i * 2
            wait_group(win_a, sem_a)
            fetch_group(g0 + 1, win_b, sem_b)
            s = process(g0, win_a, s)
            wait_group(win_b, sem_b)
            fetch_group(g0 + 2, win_a, sem_a)
            s = process(g0 + 1, win_b, s)
            return s

        s = lax.fori_loop(0, 16, outer, jnp.int32(0))
        wait_group(win_a, sem_a)

        @pl.when((s & 3) != 0)
        def _():
            dst = pl.multiple_of(region + (s >> 2) * 128, 128)
            pltpu.sync_copy(rstage, rows_hbm.at[pl.ds(dst, 128)])

        pltpu.sync_copy(pext, pos_hbm.at[pl.ds(wid * _B, _B)])

        for j in range(8):
            plsc.store_scatter(cvec, [j * 16 + _iota16()], _fill(s))
        pltpu.sync_copy(cvec, cnt_hbm.at[pl.ds(wid * 128, 128)])

    return k1


@functools.lru_cache(maxsize=None)
def _build_k2():
    info = plsc.get_sparse_core_info()
    nc = info.num_cores
    mesh = plsc.VectorSubcoreMesh(core_axis_name="c", subcore_axis_name="s")

    @functools.partial(
        pl.kernel,
        mesh=mesh,
        out_type=jax.ShapeDtypeStruct((_B, _D), jnp.float32),
        scratch_types=[
            pltpu.VMEM((128,), jnp.int32),
            pltpu.VMEM((128,), jnp.int32),
            pltpu.VMEM((128, _D), jnp.float32),
            pltpu.SemaphoreType.DMA,
        ],
        compiler_params=pltpu.CompilerParams(
            use_tc_tiling_on_sc=False, needs_layout_passes=False
        ),
    )
    def k2(cnt_hbm, pos_hbm, rows_hbm, out_hbm, cvec, idx64, stage, sem):
        wid = lax.axis_index("s") * nc + lax.axis_index("c")
        pltpu.sync_copy(cnt_hbm.at[pl.ds(wid * 128, 128)], cvec)
        cnt = plsc.load_gather(cvec, [_fill(0)])[0]
        nch = (cnt + 127) // 128

        def cb(c, _):
            row0 = wid * _B + c * 128
            pltpu.sync_copy(pos_hbm.at[pl.ds(row0, 128)], idx64)
            pltpu.sync_copy(rows_hbm.at[pl.ds(row0, 128), :], stage)
            rem = cnt - c * 128

            @pl.when(rem < 128)
            def _():
                p0 = plsc.load_gather(idx64, [_fill(0)])[0]
                for j in range(8):
                    lanes = j * 16 + _iota16()
                    vv = plsc.load_gather(idx64, [lanes])
                    sel = jnp.where(lanes >= rem, _fill(p0), vv)
                    plsc.store_scatter(idx64, [lanes], sel)
                r0a = plsc.load_gather(stage, [_fill(0), _iota16()])
                r0b = plsc.load_gather(stage, [_fill(0), _iota16() + 16])
                for r in range(1, 128):
                    @pl.when(r >= rem)
                    def _():
                        plsc.store_scatter(stage, [_fill(r), _iota16()], r0a)
                        plsc.store_scatter(stage, [_fill(r), _iota16() + 16], r0b)

            pltpu.async_copy(stage, out_hbm.at[idx64], sem).wait()
            return ()

        lax.fori_loop(0, nch, cb, ())

    return k2


def kernel(instance_ids, embedding_instance_weight):
    k1 = _build_k1()
    k2 = _build_k2()
    rows1, pos1, cnts = k1(
        instance_ids.astype(jnp.int32), embedding_instance_weight.T
    )
    rows2 = rows1.reshape(32 * _B, _D)
    return k2(cnts, pos1, rows2)
